# baseline (speedup 1.0000x reference)
import functools

import jax
import jax.numpy as jnp
from jax import lax
from jax.experimental import pallas as pl
from jax.experimental.pallas import tpu as pltpu
from jax.experimental.pallas import tpu_sc as plsc

NUM_FIELDS = 26
VOCAB = 100000
EMB_DIM = 32
BATCH = 16384

NC, NS, L = 2, 16, 16
NW = NC * NS
B_PER_W = BATCH // NW
CHUNK = 128
CHUNKS_PER_FIELD = B_PER_W // CHUNK
NCHUNK = NUM_FIELDS * CHUNKS_PER_FIELD
NBUF = 4
SLICES_PER_CHUNK = CHUNK * EMB_DIM // L


def _emb_body(x_hbm, tables_hbm, out_hbm, idx_v, bufs, acc, *sems):
    wid = lax.axis_index("s") * NC + lax.axis_index("c")

    pltpu.sync_copy(x_hbm.at[wid], idx_v)

    def fix_and_issue(t, b):
        off = (t // CHUNKS_PER_FIELD) * VOCAB
        for s in range(CHUNK // L):
            sl = pl.ds(s * L, L)
            idx_v[t, sl] = idx_v[t, sl] + off
        pltpu.async_copy(tables_hbm.at[idx_v.at[t]], bufs.at[b], sems[b])

    for b in range(NBUF):
        fix_and_issue(b, b)

    zeros = jnp.zeros((L,), jnp.float32)

    @plsc.parallel_loop(0, B_PER_W * EMB_DIM, step=L)
    def _(i):
        acc[pl.ds(i, L)] = zeros

    def group(g, _):
        for b in range(NBUF):
            t = g * NBUF + b
            pltpu.make_async_copy(
                tables_hbm.at[idx_v.at[t]], bufs.at[b], sems[b]).wait()

            @plsc.parallel_loop(0, CHUNK, step=1, unroll=8)
            def _(r):
                for c in range(0, EMB_DIM, L):
                    plsc.addupdate(acc.at[pl.ds((b * CHUNK + r) * EMB_DIM + c, L)],
                                   bufs[b, r, pl.ds(c, L)])

            @pl.when(t + NBUF < NCHUNK)
            def _():
                fix_and_issue(t + NBUF, b)
        return None

    lax.fori_loop(0, NCHUNK // NBUF, group, None)

    pltpu.sync_copy(acc, out_hbm.at[pl.ds(wid * B_PER_W * EMB_DIM, B_PER_W * EMB_DIM)])


@jax.jit
def _emb_call(x_prep, tables_flat):
    mesh = plsc.VectorSubcoreMesh(core_axis_name="c", subcore_axis_name="s",
                                  num_cores=NC, num_subcores=NS)
    f = pl.kernel(
        _emb_body,
        out_type=jax.ShapeDtypeStruct((BATCH * EMB_DIM,), jnp.float32),
        mesh=mesh,
        scratch_types=[
            pltpu.VMEM((NCHUNK, CHUNK), jnp.int32),
            pltpu.VMEM((NBUF, CHUNK, EMB_DIM), jnp.float32),
            pltpu.VMEM((B_PER_W * EMB_DIM,), jnp.float32),
        ] + [pltpu.SemaphoreType.DMA] * NBUF,
        compiler_params=pltpu.CompilerParams(use_tc_tiling_on_sc=False),
    )
    return f(x_prep, tables_flat)


def kernel(g, x, tables):
    del g
    x_prep = (x.astype(jnp.int32).T
              .reshape(NUM_FIELDS, NW, CHUNKS_PER_FIELD, CHUNK)
              .transpose(1, 0, 2, 3)
              .reshape(NW, NCHUNK, CHUNK))
    tables_flat = tables.reshape(NUM_FIELDS * VOCAB, EMB_DIM)
    out = _emb_call(x_prep, tables_flat)
    return out.reshape(BATCH, EMB_DIM)


# speedup vs baseline: 5.5740x; 5.5740x over previous
import jax
import jax.numpy as jnp
from jax import lax
from jax.experimental import pallas as pl
from jax.experimental.pallas import tpu as pltpu
from jax.experimental.pallas import tpu_sc as plsc

NUM_FIELDS = 26
VOCAB = 100000
EMB_DIM = 32
BATCH = 16384

NC, NS, L = 2, 16, 16
NW = NC * NS
IDX_CHUNK = 4096
NCH = BATCH // IDX_CHUNK


def _emb_body(x_hbm, tables_hbm, out_hbm, row_v, idx_v, acc, sem_row, sem_i0, sem_i1):
    d = lax.axis_index("s") * NC + lax.axis_index("c")
    sem_idx = (sem_i0, sem_i1)

    def idx_src(k):
        return x_hbm.at[k // NCH, pl.ds((k % NCH) * IDX_CHUNK, IDX_CHUNK)]

    pltpu.async_copy(idx_src(0), idx_v.at[0], sem_i0)
    pltpu.async_copy(idx_src(1), idx_v.at[1], sem_i1)
    pltpu.async_copy(tables_hbm.at[0, d], row_v, sem_row)

    zeros = jnp.zeros((L,), jnp.float32)

    @plsc.parallel_loop(0, BATCH, step=L)
    def _(i):
        acc[pl.ds(i, L)] = zeros

    def field(f, _):
        pltpu.make_async_copy(tables_hbm.at[f, d], row_v, sem_row).wait()
        for c in range(NCH):
            s = c % 2
            k = f * NCH + c
            pltpu.make_async_copy(idx_src(k), idx_v.at[s], sem_idx[s]).wait()

            @plsc.parallel_loop(0, IDX_CHUNK, step=L, unroll=8)
            def _(i):
                vals = plsc.load_gather(row_v, [idx_v[s, pl.ds(i, L)]])
                plsc.addupdate(acc.at[pl.ds(c * IDX_CHUNK + i, L)], vals)

            @pl.when(k + 2 < NUM_FIELDS * NCH)
            def _():
                pltpu.async_copy(idx_src(k + 2), idx_v.at[s], sem_idx[s])

        @pl.when(f + 1 < NUM_FIELDS)
        def _():
            pltpu.async_copy(tables_hbm.at[f + 1, d], row_v, sem_row)
        return None

    lax.fori_loop(0, NUM_FIELDS, field, None)

    pltpu.sync_copy(acc, out_hbm.at[d])


@jax.jit
def _emb_call(x_t, tables_t):
    mesh = plsc.VectorSubcoreMesh(core_axis_name="c", subcore_axis_name="s",
                                  num_cores=NC, num_subcores=NS)
    f = pl.kernel(
        _emb_body,
        out_type=jax.ShapeDtypeStruct((EMB_DIM, BATCH), jnp.float32),
        mesh=mesh,
        scratch_types=[
            pltpu.VMEM((VOCAB,), jnp.float32),
            pltpu.VMEM((2, IDX_CHUNK), jnp.int32),
            pltpu.VMEM((BATCH,), jnp.float32),
            pltpu.SemaphoreType.DMA,
            pltpu.SemaphoreType.DMA,
            pltpu.SemaphoreType.DMA,
        ],
        compiler_params=pltpu.CompilerParams(use_tc_tiling_on_sc=True,
                                             needs_layout_passes=False),
    )
    return f(x_t, tables_t)


def kernel(g, x, tables):
    del g
    x_t = x.astype(jnp.int32).T
    tables_t = tables.transpose(0, 2, 1)
    out_t = _emb_call(x_t, tables_t)
    return out_t.T


# speedup vs baseline: 6.1700x; 1.1069x over previous
import jax
import jax.numpy as jnp
from jax import lax
from jax.experimental import pallas as pl
from jax.experimental.pallas import tpu as pltpu
from jax.experimental.pallas import tpu_sc as plsc

NUM_FIELDS = 26
VOCAB = 100000
EMB_DIM = 32
BATCH = 16384

NC, NS, L = 2, 16, 16
NW = NC * NS
IDX_CHUNK = 4096
NCH = BATCH // IDX_CHUNK


W = 4


def _emb_body(x_hbm, tables_hbm, out_hbm, row_v, idx_v, acc, x_sh,
              sem_row, sem_i0, sem_i1, sem_x):
    c_ax = lax.axis_index("c")
    s_ax = lax.axis_index("s")
    d = s_ax * NC + c_ax
    sem_idx = (sem_i0, sem_i1)

    @pl.when(s_ax == 0)
    def _():
        pltpu.async_copy(x_hbm.at[pl.ds(0, W)], x_sh, sem_x)

    pltpu.async_copy(tables_hbm.at[0, d], row_v, sem_row)

    @pl.when(s_ax == 0)
    def _():
        pltpu.make_async_copy(x_hbm.at[pl.ds(0, W)], x_sh, sem_x).wait()

    plsc.subcore_barrier()

    def idx_src(k):
        return x_sh.at[(k // NCH) % W, pl.ds((k % NCH) * IDX_CHUNK, IDX_CHUNK)]

    def refill(f):
        return (x_hbm.at[f], x_sh.at[f % W], sem_x)

    pltpu.async_copy(idx_src(0), idx_v.at[0], sem_i0)
    pltpu.async_copy(idx_src(1), idx_v.at[1], sem_i1)

    zeros = jnp.zeros((L,), jnp.float32)

    @plsc.parallel_loop(0, BATCH, step=L)
    def _(i):
        acc[pl.ds(i, L)] = zeros

    def field(f, _):
        pltpu.make_async_copy(tables_hbm.at[f, d], row_v, sem_row).wait()
        for c in range(NCH):
            s = c % 2
            k = f * NCH + c
            pltpu.make_async_copy(idx_src(k), idx_v.at[s], sem_idx[s]).wait()

            @plsc.parallel_loop(0, IDX_CHUNK, step=L, unroll=8)
            def _(i):
                vals = plsc.load_gather(row_v, [idx_v[s, pl.ds(i, L)]])
                plsc.addupdate(acc.at[pl.ds(c * IDX_CHUNK + i, L)], vals)

            @pl.when(k + 2 < NUM_FIELDS * NCH)
            def _():
                pltpu.async_copy(idx_src(k + 2), idx_v.at[s], sem_idx[s])

        @pl.when(f + 1 < NUM_FIELDS)
        def _():
            pltpu.async_copy(tables_hbm.at[f + 1, d], row_v, sem_row)

        @pl.when((s_ax == 0) & (f >= 1) & (f + W - 1 < NUM_FIELDS))
        def _():
            src, dst, sem = refill(f + W - 1)
            pltpu.make_async_copy(src, dst, sem).wait()

        plsc.subcore_barrier()

        @pl.when((s_ax == 0) & (f + W < NUM_FIELDS))
        def _():
            pltpu.async_copy(*refill(f + W))
        return None

    lax.fori_loop(0, NUM_FIELDS, field, None)

    pltpu.sync_copy(acc, out_hbm.at[d])


@jax.jit
def _emb_call(x_t, tables_t):
    mesh = plsc.VectorSubcoreMesh(core_axis_name="c", subcore_axis_name="s",
                                  num_cores=NC, num_subcores=NS)
    f = pl.kernel(
        _emb_body,
        out_type=jax.ShapeDtypeStruct((EMB_DIM, BATCH), jnp.float32),
        mesh=mesh,
        scratch_types=[
            pltpu.VMEM((VOCAB,), jnp.float32),
            pltpu.VMEM((2, IDX_CHUNK), jnp.int32),
            pltpu.VMEM((BATCH,), jnp.float32),
            pltpu.VMEM_SHARED((W, BATCH), jnp.int32),
            pltpu.SemaphoreType.DMA,
            pltpu.SemaphoreType.DMA,
            pltpu.SemaphoreType.DMA,
            pltpu.SemaphoreType.DMA,
        ],
        compiler_params=pltpu.CompilerParams(use_tc_tiling_on_sc=True,
                                             needs_layout_passes=False),
    )
    return f(x_t, tables_t)


def kernel(g, x, tables):
    del g
    x_t = x.astype(jnp.int32).T
    tables_t = tables.transpose(0, 2, 1)
    out_t = _emb_call(x_t, tables_t)
    return out_t.T
